# Initial kernel scaffold; baseline (speedup 1.0000x reference)
import jax
import jax.numpy as jnp
from jax.experimental import pallas as pl


def kernel(x1, edge_index1, edge_attr1, x2, edge_index2, edge_attr2, Wn1, Ws1, b1, Wn2, Ws2, b2, Wc1, bc1, Wc2, bc2, Wc3, bc3):
    raise NotImplementedError("write your pallas kernel here")


# speedup vs baseline: 11.9745x; 11.9745x over previous
import functools

import jax
import jax.numpy as jnp
from jax import lax
from jax.experimental import pallas as pl
from jax.experimental.pallas import tpu as pltpu

N = 200
E = 12800
NFEAT = 512
NHID = 256
NCLASS = 128

EBLK = 256
NBLKS = E // EBLK


def _tdot(a, b):
    return lax.dot_general(a, b, (((0,), (0,)), ((), ())),
                           preferred_element_type=jnp.float32)


def _build_adj(ei_ref, ea_ref):
    def body(b, carry):
        acc, cnt = carry
        src = ei_ref[0, pl.ds(b * EBLK, EBLK)]
        dst = ei_ref[1, pl.ds(b * EBLK, EBLK)]
        ew = ea_ref[0, pl.ds(b * EBLK, EBLK)]
        node_iota = lax.broadcasted_iota(jnp.int32, (EBLK, N), 1)
        d_oh = (dst[:, None] == node_iota).astype(jnp.float32)
        s_oh = (src[:, None] == node_iota).astype(jnp.float32) * ew[:, None]
        acc = acc + _tdot(d_oh, s_oh)
        cnt = cnt + jnp.sum(d_oh, axis=0)[:, None]
        return acc, cnt

    acc0 = jnp.zeros((N, N), jnp.float32)
    cnt0 = jnp.zeros((N, 1), jnp.float32)
    return lax.fori_loop(0, NBLKS, body, (acc0, cnt0))


def _branch(x_ref, ei_ref, ea_ref, Wn1_ref, Ws1_ref, b1_ref, Wn2_ref,
            Ws2_ref, b2_ref):
    A, cnt = _build_adj(ei_ref, ea_ref)
    inv = 1.0 / jnp.maximum(cnt, 1.0)
    x = x_ref[...]
    agg1 = jnp.dot(A, x, preferred_element_type=jnp.float32) * inv
    h = jax.nn.relu(jnp.dot(agg1, Wn1_ref[...], preferred_element_type=jnp.float32)
                    + jnp.dot(x, Ws1_ref[...], preferred_element_type=jnp.float32)
                    + b1_ref[...])
    agg2 = jnp.dot(A, h, preferred_element_type=jnp.float32) * inv
    o = (jnp.dot(agg2, Wn2_ref[...], preferred_element_type=jnp.float32)
         + jnp.dot(h, Ws2_ref[...], preferred_element_type=jnp.float32)
         + b2_ref[...])
    return o


def _classifier(o, Wc1_ref, bc1_ref, Wc2_ref, bc2_ref, Wc3_ref, bc3_ref):
    t = jax.nn.relu(_tdot(o, Wc1_ref[...]) + bc1_ref[...])
    t = jax.nn.relu(jnp.dot(t, Wc2_ref[...], preferred_element_type=jnp.float32)
                    + bc2_ref[...])
    return jnp.dot(t, Wc3_ref[...], preferred_element_type=jnp.float32) + bc3_ref[...]


def _fused_kernel(x1_ref, ei1_ref, ea1_ref, x2_ref, ei2_ref, ea2_ref,
                  Wn1_ref, Ws1_ref, b1_ref, Wn2_ref, Ws2_ref, b2_ref,
                  Wc1_ref, bc1_ref, Wc2_ref, bc2_ref, Wc3_ref, bc3_ref,
                  out1_ref, out2_ref):
    o1 = _branch(x1_ref, ei1_ref, ea1_ref, Wn1_ref, Ws1_ref, b1_ref,
                 Wn2_ref, Ws2_ref, b2_ref)
    o2 = _branch(x2_ref, ei2_ref, ea2_ref, Wn1_ref, Ws1_ref, b1_ref,
                 Wn2_ref, Ws2_ref, b2_ref)
    out1_ref[...] = _classifier(o1, Wc1_ref, bc1_ref, Wc2_ref, bc2_ref,
                                Wc3_ref, bc3_ref)
    out2_ref[...] = _classifier(o2, Wc1_ref, bc1_ref, Wc2_ref, bc2_ref,
                                Wc3_ref, bc3_ref)


@jax.jit
def kernel(x1, edge_index1, edge_attr1, x2, edge_index2, edge_attr2,
           Wn1, Ws1, b1, Wn2, Ws2, b2, Wc1, bc1, Wc2, bc2, Wc3, bc3):
    out1, out2 = pl.pallas_call(
        _fused_kernel,
        out_shape=(
            jax.ShapeDtypeStruct((NCLASS, 10), jnp.float32),
            jax.ShapeDtypeStruct((NCLASS, 10), jnp.float32),
        ),
    )(x1, edge_index1, edge_attr1.reshape(1, E),
      x2, edge_index2, edge_attr2.reshape(1, E),
      Wn1, Ws1, b1.reshape(1, NHID), Wn2, Ws2, b2.reshape(1, NCLASS),
      Wc1, bc1.reshape(1, 100), Wc2, bc2.reshape(1, 50), Wc3,
      bc3.reshape(1, 10))
    return out1, out2


# speedup vs baseline: 14.2837x; 1.1928x over previous
import functools

import jax
import jax.numpy as jnp
from jax import lax
from jax.experimental import pallas as pl
from jax.experimental.pallas import tpu as pltpu
from jax.experimental.pallas import tpu_sc as plsc

N = 200
E = 12800
NFEAT = 512
NHID = 256
NCLASS = 128

SC_W = 256
SC_SZ = N * SC_W
EPT = E // 16
IDX_ROWS = 13
NVAL = IDX_ROWS * 128
DUMP = 255
SLICE = SC_SZ // 16

_sc_mesh = plsc.VectorSubcoreMesh(core_axis_name="c", subcore_axis_name="s")


@functools.partial(
    pl.kernel,
    mesh=_sc_mesh,
    out_type=(
        jax.ShapeDtypeStruct((SC_SZ,), jnp.float32),
        jax.ShapeDtypeStruct((SC_SZ,), jnp.float32),
    ),
    scratch_types=[
        pltpu.VMEM((EPT,), jnp.int32),
        pltpu.VMEM((EPT,), jnp.int32),
        pltpu.VMEM((NVAL,), jnp.float32),
        [pltpu.VMEM((128,), jnp.int32) for _ in range(IDX_ROWS)],
        pltpu.VMEM_SHARED((SC_SZ,), jnp.float32),
    ],
)
def _adj_sc(src1, dst1, va1, src2, dst2, va2, zeros_hbm, out1, out2,
            src_v, dst_v, vals_v, idx_refs, acc_sh):
    cid = lax.axis_index("c")
    sid = lax.axis_index("s")
    base = sid * EPT
    row0 = sid * SLICE

    def run(src_hbm, dst_hbm, va_hbm, out_hbm):
        pltpu.sync_copy(src_hbm.at[pl.ds(base, EPT)], src_v)
        pltpu.sync_copy(dst_hbm.at[pl.ds(base, EPT)], dst_v)
        pltpu.sync_copy(va_hbm.at[pl.ds(base, EPT)], vals_v.at[pl.ds(0, EPT)])
        pltpu.sync_copy(va_hbm.at[pl.ds(E + base, EPT)],
                        vals_v.at[pl.ds(EPT, EPT)])
        pltpu.sync_copy(va_hbm.at[pl.ds(2 * E, NVAL - 2 * EPT)],
                        vals_v.at[pl.ds(2 * EPT, NVAL - 2 * EPT)])
        pltpu.sync_copy(zeros_hbm.at[pl.ds(row0, SLICE)],
                        acc_sh.at[pl.ds(row0, SLICE)])
        for g in range(EPT // 16):
            s = src_v[pl.ds(g * 16, 16)]
            d = dst_v[pl.ds(g * 16, 16)]
            p = g * 16
            idx_refs[p // 128][pl.ds(p % 128, 16)] = d * SC_W + s
        for g in range(EPT // 16):
            d = dst_v[pl.ds(g * 16, 16)]
            p = EPT + g * 16
            idx_refs[p // 128][pl.ds(p % 128, 16)] = d * SC_W + 200
        for p in range(2 * EPT, NVAL, 16):
            idx_refs[p // 128][pl.ds(p % 128, 16)] = jnp.full((16,), DUMP,
                                                              jnp.int32)
        plsc.subcore_barrier()
        for j in range(IDX_ROWS):
            pltpu.sync_copy(vals_v.at[pl.ds(j * 128, 128)],
                            acc_sh.at[idx_refs[j]], add=True)
        plsc.subcore_barrier()
        pltpu.sync_copy(acc_sh.at[pl.ds(row0, SLICE)],
                        out_hbm.at[pl.ds(row0, SLICE)])

    @pl.when(cid == 0)
    def _():
        run(src1, dst1, va1, out1)

    @pl.when(cid == 1)
    def _():
        run(src2, dst2, va2, out2)


def _tdot(a, b):
    return lax.dot_general(a, b, (((0,), (0,)), ((), ())),
                           preferred_element_type=jnp.float32)


def _mm(a, b):
    return jnp.dot(a, b, preferred_element_type=jnp.float32)


def _branch(xe_ref, buf_ref, Wn1_ref, Ws1_ref, b1_ref, Wn2_ref, Ws2_ref,
            b2_ref):
    buf = buf_ref[...]
    cnt = buf[:, 200:201]
    inv = 1.0 / jnp.maximum(cnt, 1.0)
    xe = xe_ref[...]
    x = xe[:N, :]
    agg1 = _mm(buf, xe) * inv
    h = jax.nn.relu(_mm(agg1, Wn1_ref[...]) + _mm(x, Ws1_ref[...])
                    + b1_ref[...])
    he = jnp.concatenate([h, jnp.zeros((SC_W - N, NHID), jnp.float32)],
                         axis=0)
    agg2 = _mm(buf, he) * inv
    return _mm(agg2, Wn2_ref[...]) + _mm(h, Ws2_ref[...]) + b2_ref[...]


def _classifier(o, Wc1_ref, bc1_ref, Wc2_ref, bc2_ref, Wc3_ref, bc3_ref):
    t = jax.nn.relu(_tdot(o, Wc1_ref[...]) + bc1_ref[...])
    t = jax.nn.relu(_mm(t, Wc2_ref[...]) + bc2_ref[...])
    return _mm(t, Wc3_ref[...]) + bc3_ref[...]


def _dense_kernel(x1e_ref, buf1_ref, x2e_ref, buf2_ref,
                  Wn1_ref, Ws1_ref, b1_ref, Wn2_ref, Ws2_ref, b2_ref,
                  Wc1_ref, bc1_ref, Wc2_ref, bc2_ref, Wc3_ref, bc3_ref,
                  out1_ref, out2_ref):
    o1 = _branch(x1e_ref, buf1_ref, Wn1_ref, Ws1_ref, b1_ref, Wn2_ref,
                 Ws2_ref, b2_ref)
    o2 = _branch(x2e_ref, buf2_ref, Wn1_ref, Ws1_ref, b1_ref, Wn2_ref,
                 Ws2_ref, b2_ref)
    out1_ref[...] = _classifier(o1, Wc1_ref, bc1_ref, Wc2_ref, bc2_ref,
                                Wc3_ref, bc3_ref)
    out2_ref[...] = _classifier(o2, Wc1_ref, bc1_ref, Wc2_ref, bc2_ref,
                                Wc3_ref, bc3_ref)


@jax.jit
def kernel(x1, edge_index1, edge_attr1, x2, edge_index2, edge_attr2,
           Wn1, Ws1, b1, Wn2, Ws2, b2, Wc1, bc1, Wc2, bc2, Wc3, bc3):
    ones = jnp.ones((E,), jnp.float32)
    pad0 = jnp.zeros((NVAL - 2 * EPT,), jnp.float32)
    va1 = jnp.concatenate([edge_attr1, ones, pad0])
    va2 = jnp.concatenate([edge_attr2, ones, pad0])
    zeros_hbm = jnp.zeros((SC_SZ,), jnp.float32)
    buf1_flat, buf2_flat = _adj_sc(edge_index1[0], edge_index1[1], va1,
                                   edge_index2[0], edge_index2[1], va2,
                                   zeros_hbm)
    buf1 = buf1_flat.reshape(N, SC_W)
    buf2 = buf2_flat.reshape(N, SC_W)

    xpad = jnp.zeros((SC_W - N, NFEAT), jnp.float32)
    x1e = jnp.concatenate([x1, xpad], axis=0)
    x2e = jnp.concatenate([x2, xpad], axis=0)

    out1, out2 = pl.pallas_call(
        _dense_kernel,
        out_shape=(
            jax.ShapeDtypeStruct((NCLASS, 10), jnp.float32),
            jax.ShapeDtypeStruct((NCLASS, 10), jnp.float32),
        ),
    )(x1e, buf1, x2e, buf2,
      Wn1, Ws1, b1.reshape(1, NHID), Wn2, Ws2, b2.reshape(1, NCLASS),
      Wc1, bc1.reshape(1, 100), Wc2, bc2.reshape(1, 50), Wc3,
      bc3.reshape(1, 10))
    return out1, out2


# speedup vs baseline: 16.1279x; 1.1291x over previous
import functools

import jax
import jax.numpy as jnp
from jax import lax
from jax.experimental import pallas as pl
from jax.experimental.pallas import tpu as pltpu
from jax.experimental.pallas import tpu_sc as plsc

N = 200
E = 12800
NFEAT = 512
NHID = 256
NCLASS = 128

HALF = 25600
SC_SZ = 2 * HALF
EPT = E // 16
IDX_ROWS = 13
NVAL = IDX_ROWS * 128
CNT_COL = 72
DUMP = HALF + 127
SLICE = SC_SZ // 16

_sc_mesh = plsc.VectorSubcoreMesh(core_axis_name="c", subcore_axis_name="s")


@functools.partial(
    pl.kernel,
    mesh=_sc_mesh,
    out_type=jax.ShapeDtypeStruct((2 * SC_SZ,), jnp.float32),
    scratch_types=[
        pltpu.VMEM((EPT,), jnp.int32),
        pltpu.VMEM((EPT,), jnp.int32),
        pltpu.VMEM((NVAL,), jnp.float32),
        [pltpu.VMEM((128,), jnp.int32) for _ in range(IDX_ROWS)],
        pltpu.VMEM((SLICE,), jnp.float32),
        pltpu.VMEM_SHARED((SC_SZ,), jnp.float32),
        pltpu.SemaphoreType.DMA,
    ],
)
def _adj_sc(srcS, dstS, eaS, ones_hbm, out,
            src_v, dst_v, vals_v, idx_refs, zbuf, acc_sh, sem):
    cid = lax.axis_index("c")
    sid = lax.axis_index("s")
    ebase = cid * E + sid * EPT
    row0 = sid * SLICE

    cps = [
        pltpu.async_copy(srcS.at[pl.ds(ebase, EPT)], src_v, sem),
        pltpu.async_copy(dstS.at[pl.ds(ebase, EPT)], dst_v, sem),
        pltpu.async_copy(eaS.at[pl.ds(ebase, EPT)],
                         vals_v.at[pl.ds(0, EPT)], sem),
        pltpu.async_copy(ones_hbm.at[pl.ds(0, NVAL - EPT)],
                         vals_v.at[pl.ds(EPT, NVAL - EPT)], sem),
    ]
    zero16 = jnp.zeros((16,), jnp.float32)
    for i in range(SLICE // 16):
        zbuf[pl.ds(i * 16, 16)] = zero16
    for c in cps:
        c.wait()
    pltpu.sync_copy(zbuf, acc_sh.at[pl.ds(row0, SLICE)])
    for g in range(EPT // 16):
        s = src_v[pl.ds(g * 16, 16)]
        d = dst_v[pl.ds(g * 16, 16)]
        p = g * 16
        hi = jnp.where(s >= 128, jnp.int32(HALF - 128), jnp.int32(0))
        idx_refs[p // 128][pl.ds(p % 128, 16)] = hi + d * 128 + s
    for g in range(EPT // 16):
        d = dst_v[pl.ds(g * 16, 16)]
        p = EPT + g * 16
        idx_refs[p // 128][pl.ds(p % 128, 16)] = d * 128 + (HALF + CNT_COL)
    for p in range(2 * EPT, NVAL, 16):
        idx_refs[p // 128][pl.ds(p % 128, 16)] = jnp.full((16,), DUMP,
                                                          jnp.int32)
    plsc.subcore_barrier()
    for j in range(IDX_ROWS):
        pltpu.sync_copy(vals_v.at[pl.ds(j * 128, 128)],
                        acc_sh.at[idx_refs[j]], add=True)
    plsc.subcore_barrier()
    pltpu.sync_copy(acc_sh.at[pl.ds(row0, SLICE)],
                    out.at[pl.ds(cid * SC_SZ + row0, SLICE)])


def _tdot(a, b):
    return lax.dot_general(a, b, (((0,), (0,)), ((), ())),
                           preferred_element_type=jnp.float32)


def _mm(a, b):
    return jnp.dot(a, b, preferred_element_type=jnp.float32)


def _branch(x_ref, xb_ref, buf_ref, b0, Wn1_ref, Ws1_ref, b1_ref, Wn2_ref,
            Ws2_ref, b2_ref):
    alo = buf_ref[b0:b0 + N, :]
    ahi = buf_ref[b0 + N:b0 + 2 * N, :]
    sel = (lax.broadcasted_iota(jnp.int32, (128, 1), 0)
           == CNT_COL).astype(jnp.float32)
    cnt = _mm(ahi, sel)
    inv = 1.0 / jnp.maximum(cnt, 1.0)
    x = x_ref[...]
    xa = x[:128, :]
    xb = xb_ref[...]
    agg1 = (_mm(alo, xa) + _mm(ahi, xb)) * inv
    h = jax.nn.relu(_mm(agg1, Wn1_ref[...]) + _mm(x, Ws1_ref[...])
                    + b1_ref[...])
    ha = h[:128, :]
    hb = jnp.concatenate([h[128:, :], jnp.zeros((256 - N, NHID),
                                                jnp.float32)], axis=0)
    agg2 = (_mm(alo, ha) + _mm(ahi, hb)) * inv
    return _mm(agg2, Wn2_ref[...]) + _mm(h, Ws2_ref[...]) + b2_ref[...]


def _classifier(o, Wc1_ref, bc1_ref, Wc2_ref, bc2_ref, Wc3_ref, bc3_ref):
    t = jax.nn.relu(_tdot(o, Wc1_ref[...]) + bc1_ref[...])
    t = jax.nn.relu(_mm(t, Wc2_ref[...]) + bc2_ref[...])
    return _mm(t, Wc3_ref[...]) + bc3_ref[...]


def _dense_kernel(x1_ref, xb1_ref, x2_ref, xb2_ref, buf_ref,
                  Wn1_ref, Ws1_ref, b1_ref, Wn2_ref, Ws2_ref, b2_ref,
                  Wc1_ref, bc1_ref, Wc2_ref, bc2_ref, Wc3_ref, bc3_ref,
                  out1_ref, out2_ref):
    o1 = _branch(x1_ref, xb1_ref, buf_ref, 0, Wn1_ref, Ws1_ref, b1_ref,
                 Wn2_ref, Ws2_ref, b2_ref)
    o2 = _branch(x2_ref, xb2_ref, buf_ref, 2 * N, Wn1_ref, Ws1_ref, b1_ref,
                 Wn2_ref, Ws2_ref, b2_ref)
    out1_ref[...] = _classifier(o1, Wc1_ref, bc1_ref, Wc2_ref, bc2_ref,
                                Wc3_ref, bc3_ref)
    out2_ref[...] = _classifier(o2, Wc1_ref, bc1_ref, Wc2_ref, bc2_ref,
                                Wc3_ref, bc3_ref)


@jax.jit
def kernel(x1, edge_index1, edge_attr1, x2, edge_index2, edge_attr2,
           Wn1, Ws1, b1, Wn2, Ws2, b2, Wc1, bc1, Wc2, bc2, Wc3, bc3):
    srcS = jnp.concatenate([edge_index1[0], edge_index2[0]])
    dstS = jnp.concatenate([edge_index1[1], edge_index2[1]])
    eaS = jnp.concatenate([edge_attr1, edge_attr2])
    ones_hbm = jnp.ones((NVAL - EPT,), jnp.float32)
    out_flat = _adj_sc(srcS, dstS, eaS, ones_hbm)
    buf = out_flat.reshape(4 * N, 128)

    xb1 = jnp.pad(x1[128:, :], ((0, 256 - N), (0, 0)))
    xb2 = jnp.pad(x2[128:, :], ((0, 256 - N), (0, 0)))

    out1, out2 = pl.pallas_call(
        _dense_kernel,
        out_shape=(
            jax.ShapeDtypeStruct((NCLASS, 10), jnp.float32),
            jax.ShapeDtypeStruct((NCLASS, 10), jnp.float32),
        ),
    )(x1, xb1, x2, xb2, buf,
      Wn1, Ws1, b1.reshape(1, NHID), Wn2, Ws2, b2.reshape(1, NCLASS),
      Wc1, bc1.reshape(1, 100), Wc2, bc2.reshape(1, 50), Wc3,
      bc3.reshape(1, 10))
    return out1, out2


# speedup vs baseline: 17.2047x; 1.0668x over previous
import functools

import jax
import jax.numpy as jnp
from jax import lax
from jax.experimental import pallas as pl
from jax.experimental.pallas import tpu as pltpu
from jax.experimental.pallas import tpu_sc as plsc

N = 200
E = 12800
NFEAT = 512
NHID = 256
NCLASS = 128

HALF = 25600
SC_SZ = 2 * HALF
EPT = E // 16
IDX_ROWS = 13
NVAL = IDX_ROWS * 128
CNT_COL = 72
DUMP = HALF + 127
SLICE = SC_SZ // 16

_sc_mesh = plsc.VectorSubcoreMesh(core_axis_name="c", subcore_axis_name="s")


@functools.partial(
    pl.kernel,
    mesh=_sc_mesh,
    out_type=jax.ShapeDtypeStruct((2 * SC_SZ,), jnp.float32),
    scratch_types=[
        pltpu.VMEM((EPT,), jnp.int32),
        pltpu.VMEM((EPT,), jnp.int32),
        pltpu.VMEM((NVAL,), jnp.float32),
        [pltpu.VMEM((128,), jnp.int32) for _ in range(IDX_ROWS)],
        pltpu.VMEM((SLICE,), jnp.float32),
        pltpu.VMEM_SHARED((SC_SZ,), jnp.float32),
        pltpu.SemaphoreType.DMA,
        pltpu.SemaphoreType.DMA,
    ],
)
def _adj_sc(srcS, dstS, eaS, out,
            src_v, dst_v, vals_v, idx_refs, zbuf, acc_sh, sem, sem2):
    cid = lax.axis_index("c")
    sid = lax.axis_index("s")
    ebase = cid * E + sid * EPT
    row0 = sid * SLICE

    cps = [
        pltpu.async_copy(srcS.at[pl.ds(ebase, EPT)], src_v, sem),
        pltpu.async_copy(dstS.at[pl.ds(ebase, EPT)], dst_v, sem),
        pltpu.async_copy(eaS.at[pl.ds(ebase, EPT)],
                         vals_v.at[pl.ds(0, EPT)], sem),
    ]
    zero16 = jnp.zeros((16,), jnp.float32)
    one16 = jnp.ones((16,), jnp.float32)
    for i in range(SLICE // 16):
        zbuf[pl.ds(i * 16, 16)] = zero16
    for i in range(EPT, NVAL, 16):
        vals_v[pl.ds(i, 16)] = one16
    for c in cps:
        c.wait()
    pltpu.sync_copy(zbuf, acc_sh.at[pl.ds(row0, SLICE)])
    for g in range(EPT // 16):
        s = src_v[pl.ds(g * 16, 16)]
        d = dst_v[pl.ds(g * 16, 16)]
        p = g * 16
        hi = jnp.where(s >= 128, jnp.int32(HALF - 128), jnp.int32(0))
        idx_refs[p // 128][pl.ds(p % 128, 16)] = hi + d * 128 + s
    for g in range(EPT // 16):
        d = dst_v[pl.ds(g * 16, 16)]
        p = EPT + g * 16
        idx_refs[p // 128][pl.ds(p % 128, 16)] = d * 128 + (HALF + CNT_COL)
    for p in range(2 * EPT, NVAL, 16):
        idx_refs[p // 128][pl.ds(p % 128, 16)] = jnp.full((16,), DUMP,
                                                          jnp.int32)
    plsc.subcore_barrier()
    scs = [
        pltpu.async_copy(vals_v.at[pl.ds(j * 128, 128)],
                         acc_sh.at[idx_refs[j]], sem2, add=True)
        for j in range(IDX_ROWS)
    ]
    for c in scs:
        c.wait()
    plsc.subcore_barrier()
    pltpu.sync_copy(acc_sh.at[pl.ds(row0, SLICE)],
                    out.at[pl.ds(cid * SC_SZ + row0, SLICE)])


def _tdot(a, b):
    return lax.dot_general(a, b, (((0,), (0,)), ((), ())),
                           preferred_element_type=jnp.float32)


def _mm(a, b):
    return jnp.dot(a, b, preferred_element_type=jnp.float32)


def _branch(x_ref, xb_ref, buf_ref, b0, Wn1_ref, Ws1_ref, b1_ref, Wn2_ref,
            Ws2_ref, b2_ref):
    alo = buf_ref[b0:b0 + N, :]
    ahi = buf_ref[b0 + N:b0 + 2 * N, :]
    sel = (lax.broadcasted_iota(jnp.int32, (128, 1), 0)
           == CNT_COL).astype(jnp.float32)
    cnt = _mm(ahi, sel)
    inv = 1.0 / jnp.maximum(cnt, 1.0)
    x = x_ref[...]
    xa = x[:128, :]
    xb = xb_ref[...]
    agg1 = (_mm(alo, xa) + _mm(ahi, xb)) * inv
    h = jax.nn.relu(_mm(agg1, Wn1_ref[...]) + _mm(x, Ws1_ref[...])
                    + b1_ref[...])
    ha = h[:128, :]
    hb = jnp.concatenate([h[128:, :], jnp.zeros((256 - N, NHID),
                                                jnp.float32)], axis=0)
    agg2 = (_mm(alo, ha) + _mm(ahi, hb)) * inv
    return _mm(agg2, Wn2_ref[...]) + _mm(h, Ws2_ref[...]) + b2_ref[...]


def _classifier(o, Wc1_ref, bc1_ref, Wc2_ref, bc2_ref, Wc3_ref, bc3_ref):
    t = jax.nn.relu(_tdot(o, Wc1_ref[...]) + bc1_ref[...])
    t = jax.nn.relu(_mm(t, Wc2_ref[...]) + bc2_ref[...])
    return _mm(t, Wc3_ref[...]) + bc3_ref[...]


def _dense_kernel(x1_ref, xb1_ref, x2_ref, xb2_ref, buf_ref,
                  Wn1_ref, Ws1_ref, b1_ref, Wn2_ref, Ws2_ref, b2_ref,
                  Wc1_ref, bc1_ref, Wc2_ref, bc2_ref, Wc3_ref, bc3_ref,
                  out1_ref, out2_ref):
    o1 = _branch(x1_ref, xb1_ref, buf_ref, 0, Wn1_ref, Ws1_ref, b1_ref,
                 Wn2_ref, Ws2_ref, b2_ref)
    o2 = _branch(x2_ref, xb2_ref, buf_ref, 2 * N, Wn1_ref, Ws1_ref, b1_ref,
                 Wn2_ref, Ws2_ref, b2_ref)
    out1_ref[...] = _classifier(o1, Wc1_ref, bc1_ref, Wc2_ref, bc2_ref,
                                Wc3_ref, bc3_ref)
    out2_ref[...] = _classifier(o2, Wc1_ref, bc1_ref, Wc2_ref, bc2_ref,
                                Wc3_ref, bc3_ref)


@jax.jit
def kernel(x1, edge_index1, edge_attr1, x2, edge_index2, edge_attr2,
           Wn1, Ws1, b1, Wn2, Ws2, b2, Wc1, bc1, Wc2, bc2, Wc3, bc3):
    srcS = jnp.concatenate([edge_index1[0], edge_index2[0]])
    dstS = jnp.concatenate([edge_index1[1], edge_index2[1]])
    eaS = jnp.concatenate([edge_attr1, edge_attr2])
    out_flat = _adj_sc(srcS, dstS, eaS)
    buf = out_flat.reshape(4 * N, 128)

    xb1 = jnp.pad(x1[128:, :], ((0, 256 - N), (0, 0)))
    xb2 = jnp.pad(x2[128:, :], ((0, 256 - N), (0, 0)))

    out1, out2 = pl.pallas_call(
        _dense_kernel,
        out_shape=(
            jax.ShapeDtypeStruct((NCLASS, 10), jnp.float32),
            jax.ShapeDtypeStruct((NCLASS, 10), jnp.float32),
        ),
    )(x1, xb1, x2, xb2, buf,
      Wn1, Ws1, b1.reshape(1, NHID), Wn2, Ws2, b2.reshape(1, NCLASS),
      Wc1, bc1.reshape(1, 100), Wc2, bc2.reshape(1, 50), Wc3,
      bc3.reshape(1, 10))
    return out1, out2


# speedup vs baseline: 17.4827x; 1.0162x over previous
import functools

import jax
import jax.numpy as jnp
from jax import lax
from jax.experimental import pallas as pl
from jax.experimental.pallas import tpu as pltpu
from jax.experimental.pallas import tpu_sc as plsc

N = 200
E = 12800
NFEAT = 512
NHID = 256
NCLASS = 128

HALF = 25600
SC_SZ = 2 * HALF
EPT = E // 16
IDX_ROWS = 13
NVAL = IDX_ROWS * 128
CNT_COL = 72
DUMP = HALF + 127
SLICE = SC_SZ // 16

_sc_mesh = plsc.VectorSubcoreMesh(core_axis_name="c", subcore_axis_name="s")


@functools.partial(
    pl.kernel,
    mesh=_sc_mesh,
    out_type=jax.ShapeDtypeStruct((2 * SC_SZ,), jnp.float32),
    scratch_types=[
        pltpu.VMEM((EPT,), jnp.int32),
        pltpu.VMEM((EPT,), jnp.int32),
        pltpu.VMEM((NVAL,), jnp.float32),
        [pltpu.VMEM((128,), jnp.int32) for _ in range(IDX_ROWS)],
        pltpu.VMEM((SLICE,), jnp.float32),
        pltpu.VMEM_SHARED((SC_SZ,), jnp.float32),
        pltpu.SemaphoreType.DMA,
        pltpu.SemaphoreType.DMA,
    ],
)
def _adj_sc(sdS, eaS, out,
            src_v, dst_v, vals_v, idx_refs, zbuf, acc_sh, sem, sem2):
    cid = lax.axis_index("c")
    sid = lax.axis_index("s")
    ebase = cid * E + sid * EPT
    row0 = sid * SLICE

    cps = [
        pltpu.async_copy(sdS.at[pl.ds(ebase, EPT)], src_v, sem),
        pltpu.async_copy(sdS.at[pl.ds(2 * E + ebase, EPT)], dst_v, sem),
        pltpu.async_copy(eaS.at[pl.ds(ebase, EPT)],
                         vals_v.at[pl.ds(0, EPT)], sem),
    ]
    zero16 = jnp.zeros((16,), jnp.float32)
    one16 = jnp.ones((16,), jnp.float32)
    for i in range(SLICE // 16):
        zbuf[pl.ds(i * 16, 16)] = zero16
    for i in range(EPT, NVAL, 16):
        vals_v[pl.ds(i, 16)] = one16
    for c in cps:
        c.wait()
    pltpu.sync_copy(zbuf, acc_sh.at[pl.ds(row0, SLICE)])
    for g in range(EPT // 16):
        s = src_v[pl.ds(g * 16, 16)]
        d = dst_v[pl.ds(g * 16, 16)]
        p = g * 16
        hi = jnp.where(s >= 128, jnp.int32(HALF - 128), jnp.int32(0))
        idx_refs[p // 128][pl.ds(p % 128, 16)] = hi + d * 128 + s
    for g in range(EPT // 16):
        d = dst_v[pl.ds(g * 16, 16)]
        p = EPT + g * 16
        idx_refs[p // 128][pl.ds(p % 128, 16)] = d * 128 + (HALF + CNT_COL)
    for p in range(2 * EPT, NVAL, 16):
        idx_refs[p // 128][pl.ds(p % 128, 16)] = jnp.full((16,), DUMP,
                                                          jnp.int32)
    plsc.subcore_barrier()
    scs = [
        pltpu.async_copy(vals_v.at[pl.ds(j * 128, 128)],
                         acc_sh.at[idx_refs[j]], sem2, add=True)
        for j in range(IDX_ROWS)
    ]
    for c in scs:
        c.wait()
    plsc.subcore_barrier()
    pltpu.sync_copy(acc_sh.at[pl.ds(row0, SLICE)],
                    out.at[pl.ds(cid * SC_SZ + row0, SLICE)])


def _tdot(a, b):
    return lax.dot_general(a, b, (((0,), (0,)), ((), ())),
                           preferred_element_type=jnp.float32)


def _mm(a, b):
    return jnp.dot(a, b, preferred_element_type=jnp.float32)


def _pre_kernel(x1_ref, x2_ref, Ws1_ref, b1_ref, xs1_ref, xs2_ref):
    xs1_ref[...] = _mm(x1_ref[...], Ws1_ref[...]) + b1_ref[...]
    xs2_ref[...] = _mm(x2_ref[...], Ws1_ref[...]) + b1_ref[...]


def _branch(x_ref, xb_ref, buf_ref, b0, xs_ref, Wn1_ref, Wn2_ref,
            Ws2_ref, b2_ref):
    alo = buf_ref[b0:b0 + N, :]
    ahi = buf_ref[b0 + N:b0 + 2 * N, :]
    sel = (lax.broadcasted_iota(jnp.int32, (128, 1), 0)
           == CNT_COL).astype(jnp.float32)
    cnt = _mm(ahi, sel)
    inv = 1.0 / jnp.maximum(cnt, 1.0)
    x = x_ref[...]
    xa = x[:128, :]
    xb = xb_ref[...]
    agg1 = (_mm(alo, xa) + _mm(ahi, xb)) * inv
    h = jax.nn.relu(_mm(agg1, Wn1_ref[...]) + xs_ref[...])
    ha = h[:128, :]
    hb = jnp.concatenate([h[128:, :], jnp.zeros((256 - N, NHID),
                                                jnp.float32)], axis=0)
    agg2 = (_mm(alo, ha) + _mm(ahi, hb)) * inv
    return _mm(agg2, Wn2_ref[...]) + _mm(h, Ws2_ref[...]) + b2_ref[...]


def _classifier(o, Wc1_ref, bc1_ref, Wc2_ref, bc2_ref, Wc3_ref, bc3_ref):
    t = jax.nn.relu(_tdot(o, Wc1_ref[...]) + bc1_ref[...])
    t = jax.nn.relu(_mm(t, Wc2_ref[...]) + bc2_ref[...])
    return _mm(t, Wc3_ref[...]) + bc3_ref[...]


def _dense_kernel(x1_ref, xb1_ref, x2_ref, xb2_ref, buf_ref, xs1_ref,
                  xs2_ref, Wn1_ref, Wn2_ref, Ws2_ref, b2_ref,
                  Wc1_ref, bc1_ref, Wc2_ref, bc2_ref, Wc3_ref, bc3_ref,
                  out1_ref, out2_ref):
    o1 = _branch(x1_ref, xb1_ref, buf_ref, 0, xs1_ref, Wn1_ref,
                 Wn2_ref, Ws2_ref, b2_ref)
    o2 = _branch(x2_ref, xb2_ref, buf_ref, 2 * N, xs2_ref, Wn1_ref,
                 Wn2_ref, Ws2_ref, b2_ref)
    out1_ref[...] = _classifier(o1, Wc1_ref, bc1_ref, Wc2_ref, bc2_ref,
                                Wc3_ref, bc3_ref)
    out2_ref[...] = _classifier(o2, Wc1_ref, bc1_ref, Wc2_ref, bc2_ref,
                                Wc3_ref, bc3_ref)


@jax.jit
def kernel(x1, edge_index1, edge_attr1, x2, edge_index2, edge_attr2,
           Wn1, Ws1, b1, Wn2, Ws2, b2, Wc1, bc1, Wc2, bc2, Wc3, bc3):
    sdS = jnp.concatenate([edge_index1[0], edge_index2[0],
                           edge_index1[1], edge_index2[1]])
    eaS = jnp.concatenate([edge_attr1, edge_attr2])
    out_flat = _adj_sc(sdS, eaS)
    buf = out_flat.reshape(4 * N, 128)

    xb1 = jnp.pad(x1[128:, :], ((0, 256 - N), (0, 0)))
    xb2 = jnp.pad(x2[128:, :], ((0, 256 - N), (0, 0)))

    xs1, xs2 = pl.pallas_call(
        _pre_kernel,
        out_shape=(
            jax.ShapeDtypeStruct((N, NHID), jnp.float32),
            jax.ShapeDtypeStruct((N, NHID), jnp.float32),
        ),
    )(x1, x2, Ws1, b1.reshape(1, NHID))

    out1, out2 = pl.pallas_call(
        _dense_kernel,
        out_shape=(
            jax.ShapeDtypeStruct((NCLASS, 10), jnp.float32),
            jax.ShapeDtypeStruct((NCLASS, 10), jnp.float32),
        ),
    )(x1, xb1, x2, xb2, buf, xs1, xs2,
      Wn1, Wn2, Ws2, b2.reshape(1, NCLASS),
      Wc1, bc1.reshape(1, 100), Wc2, bc2.reshape(1, 50), Wc3,
      bc3.reshape(1, 10))
    return out1, out2
